# Initial kernel scaffold; baseline (speedup 1.0000x reference)
import jax
import jax.numpy as jnp
from jax.experimental import pallas as pl


def kernel(x, edge_index, eps, W1, b1, W2, b2, gamma, beta):
    raise NotImplementedError("write your pallas kernel here")


# speedup vs baseline: 3.0296x; 3.0296x over previous
import functools

import jax
import jax.numpy as jnp
from jax import lax
from jax.experimental import pallas as pl
from jax.experimental.pallas import tpu as pltpu
from jax.experimental.pallas import tpu_sc as plsc

N_NODES = 10000
N_EDGES = 160000
EMB = 256
HID = 512
DEPTH = 3
HALF = EMB // 2

NUM_SC = 2
TILES = 16
EDGES_PER_TILE = N_EDGES // TILES
CHUNK = 80
NCHUNK = EDGES_PER_TILE // CHUNK
RPT = 632
RPT_LAST = N_NODES - (TILES - 1) * RPT


@functools.cache
def _sc_segment_sum():
  mesh = plsc.VectorSubcoreMesh(core_axis_name="c", subcore_axis_name="s",
                                num_cores=NUM_SC, num_subcores=TILES)

  @functools.partial(
      pl.kernel,
      out_type=jax.ShapeDtypeStruct((2 * N_NODES, HALF), jnp.float32),
      mesh=mesh,
      scratch_types=[
          pltpu.VMEM((CHUNK,), jnp.int32),
          pltpu.VMEM((CHUNK,), jnp.int32),
          pltpu.VMEM((CHUNK, HALF), jnp.float32),
          pltpu.VMEM_SHARED((N_NODES, HALF), jnp.float32),
          pltpu.SemaphoreType.DMA,
      ],
  )
  def seg_sum(h2, gidx, dst, zrows, out, sidx_v, didx_v, rows_v, acc, sem):
    c = lax.axis_index("c")
    s = lax.axis_index("s")

    rbase = pl.multiple_of(s * RPT, 8)

    @pl.when(s < TILES - 1)
    def _():
      pltpu.sync_copy(zrows, acc.at[pl.ds(rbase, RPT)])

    @pl.when(s == TILES - 1)
    def _():
      pltpu.sync_copy(zrows.at[pl.ds(0, RPT_LAST)],
                      acc.at[pl.ds((TILES - 1) * RPT, RPT_LAST)])

    plsc.subcore_barrier()

    ebase = c * N_EDGES + s * EDGES_PER_TILE
    dbase = s * EDGES_PER_TILE

    def chunk(k, carry):
      off = k * CHUNK
      pltpu.sync_copy(gidx.at[pl.ds(ebase + off, CHUNK)], sidx_v)
      pltpu.sync_copy(dst.at[pl.ds(dbase + off, CHUNK)], didx_v)
      pltpu.async_copy(h2.at[sidx_v], rows_v, sem).wait()
      pltpu.sync_copy(rows_v, acc.at[didx_v], add=True)
      return carry

    lax.fori_loop(0, NCHUNK, chunk, 0)
    plsc.subcore_barrier()

    obase = pl.multiple_of(c * N_NODES + rbase, 8)

    @pl.when(s < TILES - 1)
    def _():
      pltpu.sync_copy(acc.at[pl.ds(rbase, RPT)], out.at[pl.ds(obase, RPT)])

    @pl.when(s == TILES - 1)
    def _():
      pltpu.sync_copy(
          acc.at[pl.ds((TILES - 1) * RPT, RPT_LAST)],
          out.at[pl.ds(pl.multiple_of(c * N_NODES + (TILES - 1) * RPT, 8),
                       RPT_LAST)])

  return seg_sum


BR = 1000
NB = N_NODES // BR


def _tc_layer(h, agg_flat, eps_i, W1, b1, W2, b2, gamma, beta, m_prev):
  def body(eps_r, h_r, aA_r, aB_r, W1_r, b1_r, W2_r, b2_r, g_r, be_r, m_r,
           hn_r, mn_r, v_s, acc_s):
    ph = pl.program_id(0)
    b = pl.program_id(1)

    @pl.when(ph == 0)
    def _():
      @pl.when(b == 0)
      def _():
        acc_s[...] = jnp.zeros_like(acc_s)

      agg = jnp.concatenate([aA_r[...], aB_r[...]], axis=1)
      z = (1.0 + eps_r[0, 0]) * h_r[...] + agg
      u = jnp.maximum(
          jnp.dot(z, W1_r[...], preferred_element_type=jnp.float32) + b1_r[...],
          0.0)
      v = jnp.dot(u, W2_r[...], preferred_element_type=jnp.float32) + b2_r[...]
      v_s[pl.ds(b * BR, BR), :] = v
      acc_s[0:1, :] += jnp.sum(v, axis=0, keepdims=True)
      acc_s[1:2, :] += jnp.sum(v * v, axis=0, keepdims=True)

    @pl.when(ph == 1)
    def _():
      mean = acc_s[0:1, :] * (1.0 / N_NODES)
      var = acc_s[1:2, :] * (1.0 / N_NODES) - mean * mean
      inv = lax.rsqrt(var + 1e-5)
      v = v_s[pl.ds(b * BR, BR), :]
      zz = (v - mean) * (inv * g_r[...]) + be_r[...]
      hn = jnp.maximum(zz, 0.0)
      hn_r[...] = hn
      mn_r[...] = jnp.maximum(m_r[...], hn)

  def on_ph0(ph, b):
    return (jnp.where(ph == 0, b, 0), 0)

  def on_ph1(ph, b):
    return (jnp.where(ph == 1, b, 0), 0)

  const = lambda ph, b: (0, 0)

  h_new, m_new = pl.pallas_call(
      body,
      grid=(2, NB),
      in_specs=[
          pl.BlockSpec(memory_space=pltpu.SMEM),
          pl.BlockSpec((BR, EMB), on_ph0),
          pl.BlockSpec((BR, HALF), on_ph0),
          pl.BlockSpec((BR, HALF),
                       lambda ph, b: (jnp.where(ph == 0, b + NB, NB), 0)),
          pl.BlockSpec((EMB, HID), const),
          pl.BlockSpec((1, HID), const),
          pl.BlockSpec((HID, EMB), const),
          pl.BlockSpec((1, EMB), const),
          pl.BlockSpec((1, EMB), const),
          pl.BlockSpec((1, EMB), const),
          pl.BlockSpec((BR, EMB), on_ph1),
      ],
      out_specs=[
          pl.BlockSpec((BR, EMB), on_ph1),
          pl.BlockSpec((BR, EMB), on_ph1),
      ],
      out_shape=[
          jax.ShapeDtypeStruct((N_NODES, EMB), jnp.float32),
          jax.ShapeDtypeStruct((N_NODES, EMB), jnp.float32),
      ],
      scratch_shapes=[
          pltpu.VMEM((N_NODES, EMB), jnp.float32),
          pltpu.VMEM((8, EMB), jnp.float32),
      ],
  )(eps_i, h, agg_flat, agg_flat, W1, b1, W2, b2, gamma, beta, m_prev)
  return h_new, m_new


def kernel(x, edge_index, eps, W1, b1, W2, b2, gamma, beta):
  src = edge_index[0].astype(jnp.int32)
  dst = edge_index[1].astype(jnp.int32)
  gidx = jnp.concatenate([2 * src, 2 * src + 1])
  zrows = jnp.zeros((RPT, HALF), jnp.float32)

  h = x
  m = jnp.zeros_like(x)
  for i in range(DEPTH):
    agg_flat = _sc_segment_sum()(h.reshape(2 * N_NODES, HALF), gidx, dst,
                                 zrows)
    h, m = _tc_layer(h, agg_flat, eps[i].reshape(1, 1), W1[i],
                     b1[i].reshape(1, HID), W2[i], b2[i].reshape(1, EMB),
                     gamma[i].reshape(1, EMB), beta[i].reshape(1, EMB), m)
  return m
